# Initial kernel scaffold; baseline (speedup 1.0000x reference)
import jax
import jax.numpy as jnp
from jax import lax
from jax.experimental import pallas as pl
from jax.experimental.pallas import tpu as pltpu
from jax.experimental.pallas import tpu_sc as plsc

B = 4096
H = 50
D = 64
NC = 2
NS = 16
NW = NC * NS
BW = B // NW
L = 16
NG = BW // L
NV = D // L


def _body(user_hbm, item_hbm, kwt_hbm, uemb_hbm, iemb_hbm, kemb_hbm, out_hbm,
          uidx, iidx, kidxt, urows, irows, acc, kw0, nzf, rcpf, outv, sem):
  cid = lax.axis_index("c")
  sid = lax.axis_index("s")
  wid = sid * NC + cid
  base = wid * BW

  pltpu.sync_copy(user_hbm.at[pl.ds(base, BW)], uidx)
  pltpu.sync_copy(item_hbm.at[pl.ds(base, BW)], iidx)
  pltpu.sync_copy(kwt_hbm.at[:, pl.ds(base, BW)], kidxt)

  cp_u = pltpu.async_copy(uemb_hbm.at[uidx], urows, sem)
  cp_i = pltpu.async_copy(iemb_hbm.at[iidx], irows, sem)
  cp_k0 = pltpu.async_copy(kemb_hbm.at[kidxt.at[0]], acc, sem)
  pltpu.sync_copy(kemb_hbm.at[0], kw0)

  for g in range(NG):
    def cnt_body(j, a, _g=g):
      ids = kidxt[j, pl.ds(_g * L, L)]
      return a + jnp.where(ids == 0, 1.0, 0.0)
    nz = lax.fori_loop(0, H, cnt_body, jnp.zeros((L,), jnp.float32))
    nzf[pl.ds(g * L, L)] = nz
    rcpf[pl.ds(g * L, L)] = 1.0 / jnp.maximum(jnp.float32(H) - nz, 1.0)

  cp_u.wait()
  cp_i.wait()
  cp_k0.wait()

  def fire(j, c):
    pltpu.async_copy(kemb_hbm.at[kidxt.at[j]], acc, sem, add=True)
    return c
  lax.fori_loop(1, H, fire, 0)

  def drain(j, c):
    pltpu.make_async_copy(kemb_hbm.at[kidxt.at[j]], acc, sem).wait()
    return c
  lax.fori_loop(1, H, drain, 0)

  def fin(e, c):
    nz = nzf[e]
    rcp = rcpf[e]
    s = jnp.zeros((L,), jnp.float32)
    for v in range(NV):
      sl = pl.ds(v * L, L)
      ic = (acc[e, sl] - nz * kw0[sl]) * rcp
      s = s + urows[e, sl] * (irows[e, sl] + ic)
    outv[e] = jnp.sum(s)
    return c
  lax.fori_loop(0, BW, fin, 0)

  pltpu.sync_copy(outv, out_hbm.at[pl.ds(base, BW)])


_sc_call = pl.kernel(
    _body,
    out_type=jax.ShapeDtypeStruct((B,), jnp.float32),
    mesh=plsc.VectorSubcoreMesh(core_axis_name="c", subcore_axis_name="s"),
    scratch_types=[
        pltpu.VMEM((BW,), jnp.int32),
        pltpu.VMEM((BW,), jnp.int32),
        pltpu.VMEM((H, BW), jnp.int32),
        pltpu.VMEM((BW, D), jnp.float32),
        pltpu.VMEM((BW, D), jnp.float32),
        pltpu.VMEM((BW, D), jnp.float32),
        pltpu.VMEM((D,), jnp.float32),
        pltpu.VMEM((BW,), jnp.float32),
        pltpu.VMEM((BW,), jnp.float32),
        pltpu.VMEM((BW,), jnp.float32),
        pltpu.SemaphoreType.DMA,
    ],
)


@jax.jit
def kernel(user, item, keyword_ids, user_emb, item_id_emb, keyword_emb):
  kw_t = keyword_ids.astype(jnp.int32).T
  return _sc_call(user.astype(jnp.int32), item.astype(jnp.int32), kw_t,
                  user_emb, item_id_emb, keyword_emb)


# speedup vs baseline: 5.2501x; 5.2501x over previous
import jax
import jax.numpy as jnp
from jax import lax
from jax.experimental import pallas as pl
from jax.experimental.pallas import tpu as pltpu
from jax.experimental.pallas import tpu_sc as plsc

B = 4096
H = 50
D = 64
NC = 2
NS = 16
NW = NC * NS
BW = B // NW
L = 16
NG = BW // L
NV = D // L


def _body(user_hbm, item_hbm, kwt_hbm, uemb_hbm, iemb_hbm, kemb_hbm, out_hbm,
          uidx, iidx, kidxt, urows, irows, acc, kw0, nzf, rcpf, outv, sem):
  cid = lax.axis_index("c")
  sid = lax.axis_index("s")
  wid = sid * NC + cid
  base = wid * BW

  pltpu.sync_copy(user_hbm.at[pl.ds(base, BW)], uidx)
  pltpu.sync_copy(item_hbm.at[pl.ds(base, BW)], iidx)
  pltpu.sync_copy(kwt_hbm.at[:, pl.ds(base, BW)], kidxt)

  cp_u = pltpu.async_copy(uemb_hbm.at[uidx], urows, sem)
  cp_i = pltpu.async_copy(iemb_hbm.at[iidx], irows, sem)
  cp_k0 = pltpu.async_copy(kemb_hbm.at[kidxt.at[0]], acc, sem)
  pltpu.sync_copy(kemb_hbm.at[0], kw0)

  for g in range(NG):
    def cnt_body(j, a, _g=g):
      ids = kidxt[j, pl.ds(_g * L, L)]
      return a + jnp.where(ids == 0, 1.0, 0.0)
    nz = lax.fori_loop(0, H, cnt_body, jnp.zeros((L,), jnp.float32))
    nzf[pl.ds(g * L, L)] = nz
    rcpf[pl.ds(g * L, L)] = 1.0 / jnp.maximum(jnp.float32(H) - nz, 1.0)

  cp_u.wait()
  cp_i.wait()
  cp_k0.wait()

  def fire(j, c):
    pltpu.async_copy(kemb_hbm.at[kidxt.at[j]], acc, sem,
                     add=True)
    return c
  lax.fori_loop(1, H, fire, 0)

  def drain(j, c):
    pltpu.make_async_copy(kemb_hbm.at[kidxt.at[j]], acc,
                          sem).wait()
    return c
  lax.fori_loop(1, H, drain, 0)

  lane = lax.iota(jnp.int32, L)
  last = lane == (L - 1)

  def fin(e, c):
    ev = jnp.full((L,), e, jnp.int32)
    nzv = plsc.load_gather(nzf, [ev])
    rcpv = plsc.load_gather(rcpf, [ev])
    s = jnp.zeros((L,), jnp.float32)
    for v in range(NV):
      sl = pl.ds(v * L, L)
      ic = (acc[e, sl] - nzv * kw0[sl]) * rcpv
      s = s + urows[e, sl] * (irows[e, sl] + ic)
    cs = plsc.cumsum(s)
    plsc.store_scatter(outv, [ev], cs, mask=last)
    return c

  lax.fori_loop(0, BW, fin, 0)

  pltpu.sync_copy(outv, out_hbm.at[pl.ds(base, BW)])


_sc_call = pl.kernel(
    _body,
    out_type=jax.ShapeDtypeStruct((B,), jnp.float32),
    mesh=plsc.VectorSubcoreMesh(core_axis_name="c", subcore_axis_name="s"),
    scratch_types=[
        pltpu.VMEM((BW,), jnp.int32),
        pltpu.VMEM((BW,), jnp.int32),
        pltpu.VMEM((H, BW), jnp.int32),
        pltpu.VMEM((BW, D), jnp.float32),
        pltpu.VMEM((BW, D), jnp.float32),
        pltpu.VMEM((BW, D), jnp.float32),
        pltpu.VMEM((D,), jnp.float32),
        pltpu.VMEM((BW,), jnp.float32),
        pltpu.VMEM((BW,), jnp.float32),
        pltpu.VMEM((BW,), jnp.float32),
        pltpu.SemaphoreType.DMA,
    ],
    compiler_params=pltpu.CompilerParams(
        needs_layout_passes=False, use_tc_tiling_on_sc=False),
)


@jax.jit
def kernel(user, item, keyword_ids, user_emb, item_id_emb, keyword_emb):
  kw_t = keyword_ids.astype(jnp.int32).T
  return _sc_call(user.astype(jnp.int32), item.astype(jnp.int32), kw_t,
                  user_emb, item_id_emb, keyword_emb)
